# Initial kernel scaffold; baseline (speedup 1.0000x reference)
import jax
import jax.numpy as jnp
from jax.experimental import pallas as pl


def kernel(x, edge_index, edge_traj, W_ih_f, W_hh_f, b_f, W_ih_b, W_hh_b, b_b, Wc1, Wc2, Wn2a, Wn2b, Wna, Wnb):
    raise NotImplementedError("write your pallas kernel here")


# speedup vs baseline: 5.4448x; 5.4448x over previous
import functools

import jax
import jax.numpy as jnp
from jax import lax
from jax.experimental import pallas as pl
from jax.experimental.pallas import tpu as pltpu
from jax.experimental.pallas import tpu_sc as plsc


def _node_proj_body(x_ref, wc1t_ref, wc2t_ref, ait_ref, ajt_ref, p_ref, q_ref):
    y = jnp.maximum(
        jnp.dot(x_ref[...], wc1t_ref[...], preferred_element_type=jnp.float32), 0.0)
    coor = jnp.dot(y, wc2t_ref[...], preferred_element_type=jnp.float32)
    p_ref[...] = jnp.dot(coor, ait_ref[...], preferred_element_type=jnp.float32)
    q_ref[...] = jnp.dot(coor, ajt_ref[...], preferred_element_type=jnp.float32)


def _node_proj(x, wc1t, wc2t, ait, ajt):
    n, d = x.shape
    h = wc1t.shape[1]
    nb = 2000 if n % 2000 == 0 else n
    return pl.pallas_call(
        _node_proj_body,
        grid=(n // nb,),
        in_specs=[
            pl.BlockSpec((nb, d), lambda i: (i, 0)),
            pl.BlockSpec((d, h), lambda i: (0, 0)),
            pl.BlockSpec((h, h), lambda i: (0, 0)),
            pl.BlockSpec((h, h), lambda i: (0, 0)),
            pl.BlockSpec((h, h), lambda i: (0, 0)),
        ],
        out_specs=[
            pl.BlockSpec((nb, h), lambda i: (i, 0)),
            pl.BlockSpec((nb, h), lambda i: (i, 0)),
        ],
        out_shape=[
            jax.ShapeDtypeStruct((n, h), jnp.float32),
            jax.ShapeDtypeStruct((n, h), jnp.float32),
        ],
    )(x, wc1t, wc2t, ait, ajt)


def _lstm_body(tr_ref, wf_ref, wb_ref, bf_ref, bb_ref, athalf_ref, out_ref,
               hxf, hxb, cf, cb):
    b = tr_ref.shape[0]
    t_steps = wf_ref.shape[0]
    h = 128
    feat = tr_ref.shape[1]
    tr = tr_ref[...].astype(jnp.bfloat16)
    zeros_h = jnp.zeros((b, h), jnp.bfloat16)
    hxf[:, 0:h] = zeros_h
    hxb[:, 0:h] = zeros_h
    hxf[:, h:h + feat] = tr
    hxb[:, h:h + feat] = tr
    pad = jnp.zeros((b, 256 - h - feat), jnp.bfloat16)
    hxf[:, h + feat:256] = pad
    hxb[:, h + feat:256] = pad
    cf[...] = jnp.zeros((b, h), jnp.float32)
    cb[...] = jnp.zeros((b, h), jnp.float32)

    def step(t, carry):
        for hx, c, w_ref, b_ref in ((hxf, cf, wf_ref, bf_ref),
                                    (hxb, cb, wb_ref, bb_ref)):
            z = lax.dot_general(hx[...], w_ref[t], (((1,), (0,)), ((), ())),
                                preferred_element_type=jnp.float32) + b_ref[...]
            gi = 0.5 * (1.0 + jnp.tanh(0.5 * z[:, 0:h]))
            gf = 0.5 * (1.0 + jnp.tanh(0.5 * z[:, h:2 * h]))
            gg = jnp.tanh(z[:, 2 * h:3 * h])
            go = 0.5 * (1.0 + jnp.tanh(0.5 * z[:, 3 * h:4 * h]))
            cn = gf * c[...] + gi * gg
            c[...] = cn
            hx[:, 0:h] = (go * jnp.tanh(cn)).astype(jnp.bfloat16)
        return carry

    lax.fori_loop(0, t_steps, step, 0)
    cm = cf[...] + cb[...]
    out_ref[...] = jnp.dot(cm, athalf_ref[...], preferred_element_type=jnp.float32)


def _edge_lstm(traj_r, wf_packed, wb_packed, bf2, bb2, athalf):
    e, feat = traj_r.shape
    t_steps = wf_packed.shape[0]
    h = 128
    blk = 1280 if e % 1280 == 0 else e
    return pl.pallas_call(
        _lstm_body,
        grid=(e // blk,),
        in_specs=[
            pl.BlockSpec((blk, feat), lambda i: (i, 0)),
            pl.BlockSpec((t_steps, 256, 4 * h), lambda i: (0, 0, 0)),
            pl.BlockSpec((t_steps, 256, 4 * h), lambda i: (0, 0, 0)),
            pl.BlockSpec((1, 4 * h), lambda i: (0, 0)),
            pl.BlockSpec((1, 4 * h), lambda i: (0, 0)),
            pl.BlockSpec((h, h), lambda i: (0, 0)),
        ],
        out_specs=pl.BlockSpec((blk, h), lambda i: (i, 0)),
        out_shape=jax.ShapeDtypeStruct((e, h), jnp.float32),
        scratch_shapes=[
            pltpu.VMEM((blk, 256), jnp.bfloat16),
            pltpu.VMEM((blk, 256), jnp.bfloat16),
            pltpu.VMEM((blk, h), jnp.float32),
            pltpu.VMEM((blk, h), jnp.float32),
        ],
    )(traj_r, wf_packed, wb_packed, bf2, bb2, athalf)


def _sc_scatter(t_proj, p, q, src, dst):
    e = t_proj.shape[0]
    n, h = p.shape
    nw = 32
    chunk = 80
    ew = e // nw
    nchunk = ew // chunk
    n_pad = ((n + 127) // 128) * 128
    nrows = n_pad // 16
    zeros = jnp.zeros((n_pad, h), jnp.float32)

    mesh = plsc.VectorSubcoreMesh(core_axis_name="c", subcore_axis_name="s")

    @functools.partial(
        pl.kernel,
        mesh=mesh,
        out_type=jax.ShapeDtypeStruct((2, n_pad, h), jnp.float32),
        scratch_types=[
            pltpu.VMEM((chunk,), jnp.int32),
            pltpu.VMEM((chunk,), jnp.int32),
            pltpu.VMEM((chunk, h), jnp.float32),
            pltpu.VMEM((chunk, h), jnp.float32),
            pltpu.VMEM((chunk, h), jnp.float32),
            pltpu.VMEM_SHARED((n_pad, h), jnp.float32),
            pltpu.SemaphoreType.DMA,
            pltpu.SemaphoreType.DMA,
        ],
    )
    def body(tp_hbm, p_hbm, q_hbm, src_hbm, dst_hbm, z_hbm, out_hbm,
             idx_s, idx_d, rp, rq, rt, rsh, sem_p, sem_q):
        cid = lax.axis_index("c")
        sid = lax.axis_index("s")
        wid = sid * 2 + cid
        r0 = sid * nrows
        pltpu.sync_copy(z_hbm.at[pl.ds(r0, nrows)], rsh.at[pl.ds(r0, nrows)])
        plsc.subcore_barrier()

        base = wid * ew

        def do_chunk(ci, carry):
            eb = base + ci * chunk
            pltpu.sync_copy(src_hbm.at[pl.ds(eb, chunk)], idx_s)
            pltpu.sync_copy(dst_hbm.at[pl.ds(eb, chunk)], idx_d)
            cp_p = pltpu.async_copy(p_hbm.at[idx_d], rp, sem_p)
            cp_q = pltpu.async_copy(q_hbm.at[idx_s], rq, sem_q)
            pltpu.sync_copy(tp_hbm.at[pl.ds(eb, chunk)], rt)
            cp_p.wait()
            cp_q.wait()

            def do_row(r, c2):
                for k in range(h // 16):
                    sl = pl.ds(k * 16, 16)
                    v = rp[r, sl] + rq[r, sl] + rt[r, sl]
                    rt[r, sl] = jnp.maximum(v, 0.0)
                return c2

            lax.fori_loop(0, chunk, do_row, 0)
            pltpu.sync_copy(rt, rsh.at[idx_d], add=True)
            return carry

        lax.fori_loop(0, nchunk, do_chunk, 0)
        plsc.subcore_barrier()
        pltpu.sync_copy(rsh.at[pl.ds(r0, nrows)], out_hbm.at[cid, pl.ds(r0, nrows)])

    return body(t_proj, p, q, src, dst, zeros)


def _finish_body(r_ref, wn2bt_ref, wnat_ref, wnbt_ref, out_ref, acc):
    i = pl.program_id(0)
    nblk = pl.num_programs(0)
    rsum = r_ref[0] + r_ref[1]
    agg = jnp.dot(rsum, wn2bt_ref[...], preferred_element_type=jnp.float32)
    u = jnp.maximum(
        jnp.dot(agg, wnat_ref[...], preferred_element_type=jnp.float32), 0.0)
    part = jnp.sum(u, axis=0, keepdims=True)

    @pl.when(i == 0)
    def _():
        acc[...] = jnp.zeros_like(acc)

    acc[...] += part

    @pl.when(i == nblk - 1)
    def _():
        out_ref[...] = jnp.dot(acc[...], wnbt_ref[...],
                               preferred_element_type=jnp.float32)


def _finish(rp, wn2bt, wnat, wnbt):
    _, n, h = rp.shape
    o = wnbt.shape[1]
    nb = next((b for b in (2048, 2000) if n % b == 0), n)
    return pl.pallas_call(
        _finish_body,
        grid=(n // nb,),
        in_specs=[
            pl.BlockSpec((2, nb, h), lambda i: (0, i, 0)),
            pl.BlockSpec((h, h), lambda i: (0, 0)),
            pl.BlockSpec((h, h), lambda i: (0, 0)),
            pl.BlockSpec((h, o), lambda i: (0, 0)),
        ],
        out_specs=pl.BlockSpec((1, o), lambda i: (0, 0)),
        out_shape=jax.ShapeDtypeStruct((1, o), jnp.float32),
        scratch_shapes=[pltpu.VMEM((1, h), jnp.float32)],
    )(rp, wn2bt, wnat, wnbt)


def kernel(x, edge_index, edge_traj, W_ih_f, W_hh_f, b_f, W_ih_b, W_hh_b, b_b,
           Wc1, Wc2, Wn2a, Wn2b, Wna, Wnb):
    n, d = x.shape
    e = edge_index.shape[1]
    t_steps = edge_traj.shape[1]
    h = W_hh_f.shape[1]
    feat = edge_traj.shape[2]

    ai = Wn2a[:, 0:h]
    aj = Wn2a[:, h:2 * h]
    at = Wn2a[:, 2 * h:3 * h]

    wc1t = Wc1.T
    wc2t = Wc2.T
    ait = ai.T
    ajt = aj.T
    athalf = 0.5 * at.T

    def pack_dir(w_hh, w_ih, reverse):
        wp = jnp.zeros((t_steps, 256, 4 * h), jnp.float32)
        wp = wp.at[:, 0:h, :].set(jnp.broadcast_to(w_hh.T[None], (t_steps, h, 4 * h)))
        wt = w_ih.T
        for t in range(t_steps):
            u = (t_steps - 1 - t) if reverse else t
            wp = wp.at[t, h + feat * u:h + feat * u + feat, :].set(wt)
        return wp.astype(jnp.bfloat16)

    wf_packed = pack_dir(W_hh_f, W_ih_f, False)
    wb_packed = pack_dir(W_hh_b, W_ih_b, True)
    bf2 = b_f.reshape(1, 4 * h)
    bb2 = b_b.reshape(1, 4 * h)

    traj_r = edge_traj.reshape(e, t_steps * feat)
    src = edge_index[0]
    dst = edge_index[1]

    p, q = _node_proj(x, wc1t, wc2t, ait, ajt)
    t_proj = _edge_lstm(traj_r, wf_packed, wb_packed, bf2, bb2, athalf)
    rp = _sc_scatter(t_proj, p, q, src, dst)
    out2d = _finish(rp, Wn2b.T, Wna.T, Wnb.T)
    return out2d.reshape(Wnb.shape[0])
